# baseline (speedup 1.0000x reference)
import jax
import jax.numpy as jnp
from jax.experimental import pallas as pl
from jax.experimental.pallas import tpu as pltpu

_INPUT = 768
_HIDDEN = 8192
_K = 128
_TB = 128

_INT_MIN = -2147483648


def _sae_block(x_ref, wt_ref, w_ref, b1_ref, b2_ref, o_ref, pre_ref):
    step = pl.program_id(0)
    nsteps = pl.num_programs(0)
    slot = jax.lax.rem(step, 2)

    @pl.when(step < nsteps - 1)
    def _encoder():
        xc = x_ref[...] - b2_ref[...]
        pre_ref[slot] = (
            jnp.dot(xc, wt_ref[...], preferred_element_type=jnp.float32)
            + b1_ref[...]
        )

    @pl.when(step > 0)
    def _select_and_decode():
        pre = pre_ref[1 - slot]
        ones_b = jnp.full((_HIDDEN, 128), 1, jnp.bfloat16)

        def body(i, p):
            bit = jnp.left_shift(jnp.int32(1), 31 - i)
            cand_u = p | bit
            cand_s = cand_u ^ _INT_MIN
            cand_bits = jnp.where(cand_s < 0, cand_s ^ 0x7FFFFFFF, cand_s)
            cand_f = jax.lax.bitcast_convert_type(cand_bits, jnp.float32)
            mask01 = jnp.where(pre >= cand_f, 1.0, 0.0).astype(jnp.bfloat16)
            cnt = jnp.dot(mask01, ones_b,
                          preferred_element_type=jnp.float32)[:, :1]
            return jnp.where(cnt >= _K, cand_u, p)

        p = jax.lax.fori_loop(0, 32, body, jnp.zeros((_TB, 1), jnp.int32))
        t_s = p ^ _INT_MIN
        t_bits = jnp.where(t_s < 0, t_s ^ 0x7FFFFFFF, t_s)
        thresh = jax.lax.bitcast_convert_type(t_bits, jnp.float32)

        masked = jnp.where(pre >= thresh, pre, 0.0).astype(jnp.bfloat16)
        out = jnp.dot(masked, w_ref[...], preferred_element_type=jnp.float32)
        o_ref[...] = out + b2_ref[...]


def kernel(x, W, WT, b1, b2):
    tokens = x.shape[0]
    nblocks = tokens // _TB
    w_bf16 = W.astype(jnp.bfloat16)
    b1r = b1.reshape(1, _HIDDEN)
    b2r = b2.reshape(1, _INPUT)
    return pl.pallas_call(
        _sae_block,
        grid=(nblocks + 1,),
        in_specs=[
            pl.BlockSpec((_TB, _INPUT), lambda i: (jnp.minimum(i, nblocks - 1), 0)),
            pl.BlockSpec((_INPUT, _HIDDEN), lambda i: (0, 0)),
            pl.BlockSpec((_HIDDEN, _INPUT), lambda i: (0, 0)),
            pl.BlockSpec((1, _HIDDEN), lambda i: (0, 0)),
            pl.BlockSpec((1, _INPUT), lambda i: (0, 0)),
        ],
        out_specs=pl.BlockSpec((_TB, _INPUT),
                               lambda i: (jnp.maximum(i - 1, 0), 0)),
        out_shape=jax.ShapeDtypeStruct((tokens, _INPUT), jnp.float32),
        scratch_shapes=[pltpu.VMEM((2, _TB, _HIDDEN), jnp.float32)],
        compiler_params=pltpu.CompilerParams(
            dimension_semantics=("arbitrary",),
        ),
    )(x, WT, w_bf16, b1r, b2r)


# speedup vs baseline: 1.4636x; 1.4636x over previous
import jax
import jax.numpy as jnp
from jax.experimental import pallas as pl
from jax.experimental.pallas import tpu as pltpu

_INPUT = 768
_HIDDEN = 8192
_K = 128
_TB = 128
_NCHAIN = 2

_INT_MIN = -2147483648


def _key_bits(k):
    return jnp.where(k < 0, k ^ 0x7FFFFFFF, k)


def _sae_block(x_ref, wt_ref, w_ref, b1_ref, b2_ref, o_ref):
    xc = x_ref[...] - b2_ref[...]
    pre = (
        jnp.dot(xc, wt_ref[...], preferred_element_type=jnp.float32)
        + b1_ref[...]
    )
    pre_bf = pre.astype(jnp.bfloat16)

    rows = _TB // _NCHAIN
    chains = [slice(c * rows, (c + 1) * rows) for c in range(_NCHAIN)]
    pre_c = [pre[s] for s in chains]
    pre_bf_c = [pre_bf[s] for s in chains]

    def count_ge(x_f32chain, cand_f):
        m01 = jnp.where(x_f32chain >= cand_f, 1.0, 0.0)
        return jnp.sum(m01, axis=1, keepdims=True)

    def p1_body(i, ps):
        bit = jnp.left_shift(jnp.int32(1), 15 - i)
        out = []
        for c in range(_NCHAIN):
            p = ps[c]
            cu = p | bit
            t = cu ^ 0x8000
            t16 = t - jnp.where(t >= 32768, 65536, 0)
            bits16 = jnp.where(t16 < 0, t16 ^ 0x7FFF, t16)
            cand_f = jax.lax.bitcast_convert_type(
                jnp.left_shift(bits16, 16), jnp.float32)
            cand_bf = cand_f.astype(jnp.bfloat16)
            maskb = jnp.where(pre_bf_c[c] >= cand_bf,
                              jnp.bfloat16(1), jnp.bfloat16(0))
            s = maskb
            w = _HIDDEN
            while w > 128:
                w //= 2
                s = s[:, :w] + s[:, w:2 * w]
            cnt = jnp.sum(s.astype(jnp.float32), axis=1, keepdims=True)
            out.append(jnp.where(cnt >= _K, cu, p))
        return tuple(out)

    p16 = jax.lax.fori_loop(
        0, 16, p1_body,
        tuple(jnp.zeros((rows, 1), jnp.int32) for _ in range(_NCHAIN)))

    los, his = [], []
    for c in range(_NCHAIN):
        t = p16[c] ^ 0x8000
        c16 = t - jnp.where(t >= 32768, 65536, 0)
        lo_c = jnp.maximum(c16 - 1, -32768)
        hi_c = jnp.minimum(c16 + 1, 32767)
        los.append(lo_c * 65536)
        his.append(hi_c * 65536 + 65535)

    def p2_body(i, state):
        out = []
        for c in range(_NCHAIN):
            lo, hi = state[2 * c], state[2 * c + 1]
            mid = lo + jax.lax.shift_right_logical(hi - lo, 1)
            cand_f = jax.lax.bitcast_convert_type(_key_bits(mid), jnp.float32)
            cnt = count_ge(pre_c[c], cand_f)
            ge = cnt >= _K
            out.append(jnp.where(ge, mid, lo))
            out.append(jnp.where(ge, hi, mid - 1))
        return tuple(out)

    state = jax.lax.fori_loop(
        0, 18, p2_body,
        tuple(x for c in range(_NCHAIN) for x in (los[c], his[c])))

    thresh = jnp.concatenate(
        [jax.lax.bitcast_convert_type(_key_bits(state[2 * c]), jnp.float32)
         for c in range(_NCHAIN)], axis=0)

    masked = jnp.where(pre >= thresh, pre, 0.0).astype(jnp.bfloat16)
    out = jnp.dot(masked, w_ref[...], preferred_element_type=jnp.float32)
    o_ref[...] = out + b2_ref[...]


def kernel(x, W, WT, b1, b2):
    tokens = x.shape[0]
    w_bf16 = W.astype(jnp.bfloat16)
    b1r = b1.reshape(1, _HIDDEN)
    b2r = b2.reshape(1, _INPUT)
    return pl.pallas_call(
        _sae_block,
        grid=(tokens // _TB,),
        in_specs=[
            pl.BlockSpec((_TB, _INPUT), lambda i: (i, 0)),
            pl.BlockSpec((_INPUT, _HIDDEN), lambda i: (0, 0)),
            pl.BlockSpec((_HIDDEN, _INPUT), lambda i: (0, 0)),
            pl.BlockSpec((1, _HIDDEN), lambda i: (0, 0)),
            pl.BlockSpec((1, _INPUT), lambda i: (0, 0)),
        ],
        out_specs=pl.BlockSpec((_TB, _INPUT), lambda i: (i, 0)),
        out_shape=jax.ShapeDtypeStruct((tokens, _INPUT), jnp.float32),
        compiler_params=pltpu.CompilerParams(
            dimension_semantics=("arbitrary",),
        ),
    )(x, WT, w_bf16, b1r, b2r)
